# initial kernel scaffold (unmeasured)
import jax
import jax.numpy as jnp
from jax import lax
from jax.experimental import pallas as pl
from jax.experimental.pallas import tpu as pltpu

N_DEV = 16


def kernel(x, w_mat):
    m_per, k = x.shape
    _, n = w_mat.shape
    n_per = n // N_DEV
    m_tot = m_per * N_DEV

    def body(x_ref, w_ref, out_ref, y_ref, mx_ref, recv_ref, mxrecv_ref,
             dsend_sems, drecv_sems, msend_sems, mrecv_sems):
        my = lax.axis_index("i")

        barrier = pltpu.get_barrier_semaphore()
        for j in range(1, N_DEV):
            other = lax.rem(my + j, N_DEV)
            pl.semaphore_signal(barrier, inc=1, device_id=(other,),
                                device_id_type=pl.DeviceIdType.MESH)
        pl.semaphore_wait(barrier, N_DEV - 1)

        y = jnp.maximum(
            jnp.dot(x_ref[:, :], w_ref[:, :],
                    preferred_element_type=jnp.float32),
            0.0,
        )
        y_ref[:, :] = y
        mx_ref[:, :] = jnp.full((8, 128), jnp.max(y), jnp.float32)

        recv_ref[my, :, :] = y_ref[:, pl.ds(my * n_per, n_per)]
        mxrecv_ref[my, :, :] = mx_ref[:, :]

        sends = []
        for j in range(1, N_DEV):
            dst = lax.rem(my + j, N_DEV)
            d = pltpu.make_async_remote_copy(
                src_ref=y_ref.at[:, pl.ds(dst * n_per, n_per)],
                dst_ref=recv_ref.at[my],
                send_sem=dsend_sems.at[j],
                recv_sem=drecv_sems.at[my],
                device_id=(dst,),
                device_id_type=pl.DeviceIdType.MESH,
            )
            d.start()
            m = pltpu.make_async_remote_copy(
                src_ref=mx_ref,
                dst_ref=mxrecv_ref.at[my],
                send_sem=msend_sems.at[j],
                recv_sem=mrecv_sems.at[my],
                device_id=(dst,),
                device_id_type=pl.DeviceIdType.MESH,
            )
            m.start()
            sends.append((d, m))

        for j in range(1, N_DEV):
            src = lax.rem(my + N_DEV - j, N_DEV)
            rd = pltpu.make_async_remote_copy(
                src_ref=recv_ref.at[src], dst_ref=recv_ref.at[src],
                send_sem=dsend_sems.at[j], recv_sem=drecv_sems.at[src],
                device_id=(src,), device_id_type=pl.DeviceIdType.MESH,
            )
            rd.wait_recv()
            rm = pltpu.make_async_remote_copy(
                src_ref=mxrecv_ref.at[src], dst_ref=mxrecv_ref.at[src],
                send_sem=msend_sems.at[j], recv_sem=mrecv_sems.at[src],
                device_id=(src,), device_id_type=pl.DeviceIdType.MESH,
            )
            rm.wait_recv()

        for d, m in sends:
            d.wait_send()
            m.wait_send()

        amax = jnp.max(mxrecv_ref[:, :, :])
        scale = amax / 448.0
        vals = recv_ref[:, :, :].reshape(m_tot, n_per)
        q = jnp.minimum(vals * (448.0 / amax), 448.0)
        q = q.astype(jnp.float8_e4m3fn).astype(jnp.float32)
        out_ref[:, :] = q * scale

    return pl.pallas_call(
        body,
        out_shape=jax.ShapeDtypeStruct((m_tot, n_per), jnp.float32),
        in_specs=[pl.BlockSpec(memory_space=pltpu.VMEM),
                  pl.BlockSpec(memory_space=pltpu.VMEM)],
        out_specs=pl.BlockSpec(memory_space=pltpu.VMEM),
        scratch_shapes=[
            pltpu.VMEM((m_per, n), jnp.float32),
            pltpu.VMEM((8, 128), jnp.float32),
            pltpu.VMEM((N_DEV, m_per, n_per), jnp.float32),
            pltpu.VMEM((N_DEV, 8, 128), jnp.float32),
            pltpu.SemaphoreType.DMA((N_DEV,)),
            pltpu.SemaphoreType.DMA((N_DEV,)),
            pltpu.SemaphoreType.DMA((N_DEV,)),
            pltpu.SemaphoreType.DMA((N_DEV,)),
        ],
        compiler_params=pltpu.CompilerParams(collective_id=0),
    )(x, w_mat)


# baseline (device time: 43748 ns/iter reference)
import jax
import jax.numpy as jnp
from jax import lax
from jax.experimental import pallas as pl
from jax.experimental.pallas import tpu as pltpu

N_DEV = 16


def kernel(x, w_mat):
    m_per, k = x.shape
    _, n = w_mat.shape
    n_per = n // N_DEV
    m_tot = m_per * N_DEV

    def body(x_ref, w_hbm, out_ref, wbuf, send_buf, mx_ref, recv_ref,
             mxrecv_ref, wcopy_sems, dsend_sems, drecv_sems, msend_sems,
             mrecv_sems):
        my = lax.axis_index("i")

        barrier = pltpu.get_barrier_semaphore()
        for j in range(1, N_DEV):
            other = lax.rem(my + j, N_DEV)
            pl.semaphore_signal(barrier, inc=1, device_id=(other,),
                                device_id_type=pl.DeviceIdType.MESH)
        pl.semaphore_wait(barrier, N_DEV - 1)

        def w_dma(j, slot):
            dst = lax.rem(my + j, N_DEV)
            return pltpu.make_async_copy(
                w_hbm.at[:, pl.ds(dst * n_per, n_per)],
                wbuf.at[slot],
                wcopy_sems.at[slot],
            )

        w_dma(1, 0).start()
        sends = []
        mx = jnp.float32(0.0)
        for j in range(1, N_DEV + 1):
            slot = (j - 1) % 2
            w_dma(j, slot).wait()
            if j < N_DEV:
                w_dma(j + 1, 1 - slot).start()
            chunk = jnp.maximum(
                jnp.dot(x_ref[:, :], wbuf[slot],
                        preferred_element_type=jnp.float32),
                0.0,
            )
            mx = jnp.maximum(mx, jnp.max(chunk))
            if j < N_DEV:
                dst = lax.rem(my + j, N_DEV)
                send_buf[j, :, :] = chunk
                d = pltpu.make_async_remote_copy(
                    src_ref=send_buf.at[j],
                    dst_ref=recv_ref.at[my],
                    send_sem=dsend_sems.at[j],
                    recv_sem=drecv_sems.at[my],
                    device_id=(dst,),
                    device_id_type=pl.DeviceIdType.MESH,
                )
                d.start()
                sends.append(d)
            else:
                recv_ref[my, :, :] = chunk

        mx_ref[:, :] = jnp.full((8, 128), mx, jnp.float32)
        mxrecv_ref[my, :, :] = mx_ref[:, :]
        msends = []
        for j in range(1, N_DEV):
            dst = lax.rem(my + j, N_DEV)
            m = pltpu.make_async_remote_copy(
                src_ref=mx_ref,
                dst_ref=mxrecv_ref.at[my],
                send_sem=msend_sems.at[j],
                recv_sem=mrecv_sems.at[my],
                device_id=(dst,),
                device_id_type=pl.DeviceIdType.MESH,
            )
            m.start()
            msends.append(m)

        for j in range(1, N_DEV):
            src = lax.rem(my + N_DEV - j, N_DEV)
            rd = pltpu.make_async_remote_copy(
                src_ref=recv_ref.at[src], dst_ref=recv_ref.at[src],
                send_sem=dsend_sems.at[j], recv_sem=drecv_sems.at[src],
                device_id=(src,), device_id_type=pl.DeviceIdType.MESH,
            )
            rd.wait_recv()
            rm = pltpu.make_async_remote_copy(
                src_ref=mxrecv_ref.at[src], dst_ref=mxrecv_ref.at[src],
                send_sem=msend_sems.at[j], recv_sem=mrecv_sems.at[src],
                device_id=(src,), device_id_type=pl.DeviceIdType.MESH,
            )
            rm.wait_recv()

        for d in sends:
            d.wait_send()
        for m in msends:
            m.wait_send()

        amax = jnp.max(mxrecv_ref[:, :, :])
        scale = amax / 448.0
        vals = recv_ref[:, :, :].reshape(m_tot, n_per)
        q = jnp.minimum(vals * (448.0 / amax), 448.0)
        q = q.astype(jnp.float8_e4m3fn).astype(jnp.float32)
        out_ref[:, :] = q * scale

    return pl.pallas_call(
        body,
        out_shape=jax.ShapeDtypeStruct((m_tot, n_per), jnp.float32),
        in_specs=[pl.BlockSpec(memory_space=pltpu.VMEM),
                  pl.BlockSpec(memory_space=pltpu.MemorySpace.HBM)],
        out_specs=pl.BlockSpec(memory_space=pltpu.VMEM),
        scratch_shapes=[
            pltpu.VMEM((2, k, n_per), jnp.float32),
            pltpu.VMEM((N_DEV, m_per, n_per), jnp.float32),
            pltpu.VMEM((8, 128), jnp.float32),
            pltpu.VMEM((N_DEV, m_per, n_per), jnp.float32),
            pltpu.VMEM((N_DEV, 8, 128), jnp.float32),
            pltpu.SemaphoreType.DMA((2,)),
            pltpu.SemaphoreType.DMA((N_DEV,)),
            pltpu.SemaphoreType.DMA((N_DEV,)),
            pltpu.SemaphoreType.DMA((N_DEV,)),
            pltpu.SemaphoreType.DMA((N_DEV,)),
        ],
        compiler_params=pltpu.CompilerParams(collective_id=0),
    )(x, w_mat)


# device time: 29464 ns/iter; 1.4848x vs baseline; 1.4848x over previous
import jax
import jax.numpy as jnp
from jax import lax
from jax.experimental import pallas as pl
from jax.experimental.pallas import tpu as pltpu

N_DEV = 16
G = 4
SUB = 4


def kernel(x, w_mat):
    m_per, k = x.shape
    _, n = w_mat.shape
    n_per = n // N_DEV
    gc = n // G
    kq = k // SUB
    m_tot = m_per * N_DEV

    def body(x_ref, w_hbm, out_ref, wbuf, send_ref, mx_ref, recv_ref,
             mxrecv_ref, wsems, dsend_sems, drecv_sems, msend_sems,
             mrecv_sems):
        my = lax.axis_index("i")
        my_g = lax.div(my, SUB)

        barrier = pltpu.get_barrier_semaphore()
        for j in range(1, N_DEV):
            other = lax.rem(my + j, N_DEV)
            pl.semaphore_signal(barrier, inc=1, device_id=(other,),
                                device_id_type=pl.DeviceIdType.MESH)
        pl.semaphore_wait(barrier, N_DEV - 1)

        def order(i):
            return lax.rem(my_g + 1 + i, G)

        def group_dmas(i, slot):
            g = order(i)
            return [
                pltpu.make_async_copy(
                    w_hbm.at[pl.ds(t * kq, kq), pl.ds(g * gc, gc)],
                    wbuf.at[slot, pl.ds(t * kq, kq), :],
                    wsems.at[slot * SUB + t],
                )
                for t in range(SUB)
            ]

        for d in group_dmas(0, 0):
            d.start()
        for d in group_dmas(1, 1):
            d.start()

        mx = jnp.float32(0.0)
        for i in range(G):
            slot = i % 2
            g = order(i)
            for d in group_dmas(i, slot):
                d.wait()
            chunk = jnp.maximum(
                jnp.dot(x_ref[:, :], wbuf[slot],
                        preferred_element_type=jnp.float32),
                0.0,
            )
            mx = jnp.maximum(mx, jnp.max(chunk))
            cb = chunk.astype(jnp.bfloat16)
            if i + 2 < G:
                for d in group_dmas(i + 2, slot):
                    d.start()
            for t in range(SUB):
                dst = g * SUB + t
                piece = cb[:, t * n_per:(t + 1) * n_per]

                @pl.when(dst == my)
                def _(piece=piece):
                    recv_ref[my, :, :] = piece

                @pl.when(dst != my)
                def _(piece=piece, dst=dst):
                    send_ref[dst, :, :] = piece
                    pltpu.make_async_remote_copy(
                        src_ref=send_ref.at[dst],
                        dst_ref=recv_ref.at[my],
                        send_sem=dsend_sems.at[dst],
                        recv_sem=drecv_sems.at[my],
                        device_id=(dst,),
                        device_id_type=pl.DeviceIdType.MESH,
                    ).start()

        mx_ref[:, :] = jnp.full((8, 128), mx, jnp.float32)
        mxrecv_ref[my, :, :] = mx_ref[:, :]
        for j in range(1, N_DEV):
            dst = lax.rem(my + j, N_DEV)
            pltpu.make_async_remote_copy(
                src_ref=mx_ref,
                dst_ref=mxrecv_ref.at[my],
                send_sem=msend_sems.at[dst],
                recv_sem=mrecv_sems.at[my],
                device_id=(dst,),
                device_id_type=pl.DeviceIdType.MESH,
            ).start()

        for j in range(1, N_DEV):
            src = lax.rem(my + N_DEV - j, N_DEV)
            pltpu.make_async_remote_copy(
                src_ref=recv_ref.at[src], dst_ref=recv_ref.at[src],
                send_sem=dsend_sems.at[src], recv_sem=drecv_sems.at[src],
                device_id=(src,), device_id_type=pl.DeviceIdType.MESH,
            ).wait_recv()
            pltpu.make_async_remote_copy(
                src_ref=mxrecv_ref.at[src], dst_ref=mxrecv_ref.at[src],
                send_sem=msend_sems.at[src], recv_sem=mrecv_sems.at[src],
                device_id=(src,), device_id_type=pl.DeviceIdType.MESH,
            ).wait_recv()

        for j in range(1, N_DEV):
            dst = lax.rem(my + j, N_DEV)
            pltpu.make_async_remote_copy(
                src_ref=send_ref.at[dst], dst_ref=recv_ref.at[my],
                send_sem=dsend_sems.at[dst], recv_sem=drecv_sems.at[my],
                device_id=(dst,), device_id_type=pl.DeviceIdType.MESH,
            ).wait_send()
            pltpu.make_async_remote_copy(
                src_ref=mx_ref, dst_ref=mxrecv_ref.at[my],
                send_sem=msend_sems.at[dst], recv_sem=mrecv_sems.at[my],
                device_id=(dst,), device_id_type=pl.DeviceIdType.MESH,
            ).wait_send()

        amax = jnp.max(mxrecv_ref[:, :, :])
        scale = amax / 448.0
        vals = recv_ref[:, :, :].reshape(m_tot, n_per).astype(jnp.float32)
        q = jnp.minimum(vals * (448.0 / amax), 448.0)
        q = q.astype(jnp.float8_e4m3fn).astype(jnp.float32)
        out_ref[:, :] = q * scale

    return pl.pallas_call(
        body,
        out_shape=jax.ShapeDtypeStruct((m_tot, n_per), jnp.float32),
        in_specs=[pl.BlockSpec(memory_space=pltpu.VMEM),
                  pl.BlockSpec(memory_space=pltpu.MemorySpace.HBM)],
        out_specs=pl.BlockSpec(memory_space=pltpu.VMEM),
        scratch_shapes=[
            pltpu.VMEM((2, k, gc), jnp.float32),
            pltpu.VMEM((N_DEV, m_per, n_per), jnp.bfloat16),
            pltpu.VMEM((8, 128), jnp.float32),
            pltpu.VMEM((N_DEV, m_per, n_per), jnp.bfloat16),
            pltpu.VMEM((N_DEV, 8, 128), jnp.float32),
            pltpu.SemaphoreType.DMA((2 * SUB,)),
            pltpu.SemaphoreType.DMA((N_DEV,)),
            pltpu.SemaphoreType.DMA((N_DEV,)),
            pltpu.SemaphoreType.DMA((N_DEV,)),
            pltpu.SemaphoreType.DMA((N_DEV,)),
        ],
        compiler_params=pltpu.CompilerParams(collective_id=0),
    )(x, w_mat)


# device time: 29086 ns/iter; 1.5041x vs baseline; 1.0130x over previous
import jax
import jax.numpy as jnp
from jax import lax
from jax.experimental import pallas as pl
from jax.experimental.pallas import tpu as pltpu

N_DEV = 16
G = 4
SUB = 4


def kernel(x, w_mat):
    m_per, k = x.shape
    _, n = w_mat.shape
    n_per = n // N_DEV
    gc = n // G
    kq = k // SUB
    m_tot = m_per * N_DEV

    def body(x_ref, w_hbm, out_ref, wbuf, send_ref, mx_ref, recv_ref,
             mxrecv_ref, wsems, dsend_sems, drecv_sems, msend_sems,
             mrecv_sems):
        my = lax.axis_index("i")
        my_g = lax.div(my, SUB)

        def order(i):
            return lax.rem(my_g + 1 + i, G)

        def group_dmas(i, slot):
            g = order(i)
            return [
                pltpu.make_async_copy(
                    w_hbm.at[pl.ds(t * kq, kq), pl.ds(g * gc, gc)],
                    wbuf.at[slot, pl.ds(t * kq, kq), :],
                    wsems.at[slot * SUB + t],
                )
                for t in range(SUB)
            ]

        for d in group_dmas(0, 0):
            d.start()
        for d in group_dmas(1, 1):
            d.start()

        barrier = pltpu.get_barrier_semaphore()
        for j in range(1, N_DEV):
            other = lax.rem(my + j, N_DEV)
            pl.semaphore_signal(barrier, inc=1, device_id=(other,),
                                device_id_type=pl.DeviceIdType.MESH)
        pl.semaphore_wait(barrier, N_DEV - 1)

        mx = jnp.float32(0.0)
        for i in range(G):
            slot = i % 2
            g = order(i)
            for d in group_dmas(i, slot):
                d.wait()
            chunk = jnp.maximum(
                jnp.dot(x_ref[:, :], wbuf[slot],
                        preferred_element_type=jnp.float32),
                0.0,
            )
            mx = jnp.maximum(mx, jnp.max(chunk))
            cb = chunk.astype(jnp.bfloat16)
            if i + 2 < G:
                for d in group_dmas(i + 2, slot):
                    d.start()
            for t in range(SUB):
                dst = g * SUB + t
                piece = cb[:, t * n_per:(t + 1) * n_per]

                @pl.when(dst == my)
                def _(piece=piece):
                    recv_ref[my, :, :] = piece

                @pl.when(dst != my)
                def _(piece=piece, dst=dst):
                    send_ref[dst, :, :] = piece
                    pltpu.make_async_remote_copy(
                        src_ref=send_ref.at[dst],
                        dst_ref=recv_ref.at[my],
                        send_sem=dsend_sems.at[dst],
                        recv_sem=drecv_sems.at[my],
                        device_id=(dst,),
                        device_id_type=pl.DeviceIdType.MESH,
                    ).start()

        mx_ref[:, :] = jnp.full((8, 128), mx, jnp.float32)
        mxrecv_ref[my, :, :] = mx_ref[:, :]
        for j in range(1, N_DEV):
            dst = lax.rem(my + j, N_DEV)
            pltpu.make_async_remote_copy(
                src_ref=mx_ref,
                dst_ref=mxrecv_ref.at[my],
                send_sem=msend_sems.at[dst],
                recv_sem=mrecv_sems.at[my],
                device_id=(dst,),
                device_id_type=pl.DeviceIdType.MESH,
            ).start()

        for j in range(1, N_DEV):
            src = lax.rem(my + N_DEV - j, N_DEV)
            pltpu.make_async_remote_copy(
                src_ref=recv_ref.at[src], dst_ref=recv_ref.at[src],
                send_sem=dsend_sems.at[src], recv_sem=drecv_sems.at[src],
                device_id=(src,), device_id_type=pl.DeviceIdType.MESH,
            ).wait_recv()
            pltpu.make_async_remote_copy(
                src_ref=mxrecv_ref.at[src], dst_ref=mxrecv_ref.at[src],
                send_sem=msend_sems.at[src], recv_sem=mrecv_sems.at[src],
                device_id=(src,), device_id_type=pl.DeviceIdType.MESH,
            ).wait_recv()

        for j in range(1, N_DEV):
            dst = lax.rem(my + j, N_DEV)
            pltpu.make_async_remote_copy(
                src_ref=send_ref.at[dst], dst_ref=recv_ref.at[my],
                send_sem=dsend_sems.at[dst], recv_sem=drecv_sems.at[my],
                device_id=(dst,), device_id_type=pl.DeviceIdType.MESH,
            ).wait_send()
            pltpu.make_async_remote_copy(
                src_ref=mx_ref, dst_ref=mxrecv_ref.at[my],
                send_sem=msend_sems.at[dst], recv_sem=mrecv_sems.at[my],
                device_id=(dst,), device_id_type=pl.DeviceIdType.MESH,
            ).wait_send()

        amax = jnp.max(mxrecv_ref[:, :, :])
        scale = amax / 448.0
        vals = recv_ref[:, :, :].reshape(m_tot, n_per).astype(jnp.float32)
        q = jnp.minimum(vals * (448.0 / amax), 448.0)
        q = q.astype(jnp.float8_e4m3fn).astype(jnp.float32)
        out_ref[:, :] = q * scale

    return pl.pallas_call(
        body,
        out_shape=jax.ShapeDtypeStruct((m_tot, n_per), jnp.float32),
        in_specs=[pl.BlockSpec(memory_space=pltpu.VMEM),
                  pl.BlockSpec(memory_space=pltpu.MemorySpace.HBM)],
        out_specs=pl.BlockSpec(memory_space=pltpu.VMEM),
        scratch_shapes=[
            pltpu.VMEM((2, k, gc), jnp.float32),
            pltpu.VMEM((N_DEV, m_per, n_per), jnp.bfloat16),
            pltpu.VMEM((8, 128), jnp.float32),
            pltpu.VMEM((N_DEV, m_per, n_per), jnp.bfloat16),
            pltpu.VMEM((N_DEV, 8, 128), jnp.float32),
            pltpu.SemaphoreType.DMA((2 * SUB,)),
            pltpu.SemaphoreType.DMA((N_DEV,)),
            pltpu.SemaphoreType.DMA((N_DEV,)),
            pltpu.SemaphoreType.DMA((N_DEV,)),
            pltpu.SemaphoreType.DMA((N_DEV,)),
        ],
        compiler_params=pltpu.CompilerParams(collective_id=0),
    )(x, w_mat)
